# baseline (speedup 1.0000x reference)
import jax
import jax.numpy as jnp
from jax.experimental import pallas as pl
from jax.experimental.pallas import tpu as pltpu

_N_EDGES = 8192
_N_NODES = 2048
_N_TRI = 4096
_F = 128

_JB = 512
_IB = 512
_XB = 1024


def _xw_kernel(x_ref, w0_ref, w1_ref, w2_ref, xw0_ref, xw1_ref, xw2_ref):
    x = x_ref[...]
    xw0_ref[...] = jnp.dot(x, w0_ref[...],
                           preferred_element_type=jnp.float32).astype(jnp.bfloat16)
    xw1_ref[...] = jnp.dot(x, w1_ref[...], preferred_element_type=jnp.float32)
    xw2_ref[...] = jnp.dot(x, w2_ref[...],
                           preferred_element_type=jnp.float32).astype(jnp.bfloat16)


def _d2_kernel(b2_ref, xw2_ref, out_ref):
    j = pl.program_id(0)
    b = b2_ref[...].astype(jnp.bfloat16)
    t = jax.lax.dot_general(b, xw2_ref[...], (((0,), (0,)), ((), ())),
                            preferred_element_type=jnp.float32)
    d = jnp.dot(b, t.astype(jnp.bfloat16), preferred_element_type=jnp.float32)

    @pl.when(j == 0)
    def _():
        out_ref[...] = d

    @pl.when(j != 0)
    def _():
        out_ref[...] += d


def _d0_kernel(b1_ref, xw0_ref, d2_ref, xw1_ref, out_ref):
    i = pl.program_id(0)
    n_steps = pl.num_programs(0)
    b = b1_ref[...].astype(jnp.bfloat16)
    n = jnp.dot(b, xw0_ref[...], preferred_element_type=jnp.float32)
    d = jax.lax.dot_general(b, n.astype(jnp.bfloat16), (((0,), (0,)), ((), ())),
                            preferred_element_type=jnp.float32)

    @pl.when(i == 0)
    def _():
        out_ref[...] = d

    @pl.when(i != 0)
    def _():
        out_ref[...] += d

    @pl.when(i == n_steps - 1)
    def _():
        out_ref[...] = jnp.maximum(out_ref[...] + d2_ref[...] + xw1_ref[...], 0.0)


def kernel(x, B1, B2, W0, W1, W2):
    f32 = jnp.float32
    bf16 = jnp.bfloat16

    xw0, xw1, xw2 = pl.pallas_call(
        _xw_kernel,
        grid=(_N_EDGES // _XB,),
        in_specs=[
            pl.BlockSpec((_XB, _F), lambda i: (i, 0)),
            pl.BlockSpec((_F, _F), lambda i: (0, 0)),
            pl.BlockSpec((_F, _F), lambda i: (0, 0)),
            pl.BlockSpec((_F, _F), lambda i: (0, 0)),
        ],
        out_specs=[
            pl.BlockSpec((_XB, _F), lambda i: (i, 0)),
            pl.BlockSpec((_XB, _F), lambda i: (i, 0)),
            pl.BlockSpec((_XB, _F), lambda i: (i, 0)),
        ],
        out_shape=[
            jax.ShapeDtypeStruct((_N_EDGES, _F), bf16),
            jax.ShapeDtypeStruct((_N_EDGES, _F), f32),
            jax.ShapeDtypeStruct((_N_EDGES, _F), bf16),
        ],
        compiler_params=pltpu.CompilerParams(
            dimension_semantics=("arbitrary",)),
    )(x, W0, W1, W2)

    d2 = pl.pallas_call(
        _d2_kernel,
        grid=(_N_TRI // _JB,),
        in_specs=[
            pl.BlockSpec((_N_EDGES, _JB), lambda j: (0, j)),
            pl.BlockSpec((_N_EDGES, _F), lambda j: (0, 0)),
        ],
        out_specs=pl.BlockSpec((_N_EDGES, _F), lambda j: (0, 0)),
        out_shape=jax.ShapeDtypeStruct((_N_EDGES, _F), f32),
        compiler_params=pltpu.CompilerParams(
            dimension_semantics=("arbitrary",)),
    )(B2, xw2)

    out = pl.pallas_call(
        _d0_kernel,
        grid=(_N_NODES // _IB,),
        in_specs=[
            pl.BlockSpec((_IB, _N_EDGES), lambda i: (i, 0)),
            pl.BlockSpec((_N_EDGES, _F), lambda i: (0, 0)),
            pl.BlockSpec((_N_EDGES, _F), lambda i: (0, 0)),
            pl.BlockSpec((_N_EDGES, _F), lambda i: (0, 0)),
        ],
        out_specs=pl.BlockSpec((_N_EDGES, _F), lambda i: (0, 0)),
        out_shape=jax.ShapeDtypeStruct((_N_EDGES, _F), f32),
        compiler_params=pltpu.CompilerParams(
            dimension_semantics=("arbitrary",)),
    )(B1, xw0, d2, xw1)

    return out


# speedup vs baseline: 1.3743x; 1.3743x over previous
import jax
import jax.numpy as jnp
from jax.experimental import pallas as pl
from jax.experimental.pallas import tpu as pltpu

_N_EDGES = 8192
_N_NODES = 2048
_N_TRI = 4096
_F = 128

_JB = 256
_IB = 256
_XB = 1024


def _xw_kernel(x_ref, w0_ref, w1_ref, w2_ref, xw0_ref, xw1_ref, xw2_ref):
    x = x_ref[...]
    xw0_ref[...] = jnp.dot(x, w0_ref[...],
                           preferred_element_type=jnp.float32).astype(jnp.bfloat16)
    xw1_ref[...] = jnp.dot(x, w1_ref[...], preferred_element_type=jnp.float32)
    xw2_ref[...] = jnp.dot(x, w2_ref[...],
                           preferred_element_type=jnp.float32).astype(jnp.bfloat16)


def _d2_kernel(b2_ref, xw2_ref, out_ref):
    j = pl.program_id(0)
    b = b2_ref[...].astype(jnp.bfloat16)
    t = jax.lax.dot_general(b, xw2_ref[...], (((0,), (0,)), ((), ())),
                            preferred_element_type=jnp.float32)
    d = jnp.dot(b, t.astype(jnp.bfloat16), preferred_element_type=jnp.float32)

    @pl.when(j == 0)
    def _():
        out_ref[...] = d

    @pl.when(j != 0)
    def _():
        out_ref[...] += d


def _d0_kernel(b1_ref, xw0_ref, d2_ref, xw1_ref, out_ref):
    i = pl.program_id(0)
    n_steps = pl.num_programs(0)
    b = b1_ref[...].astype(jnp.bfloat16)
    n = jnp.dot(b, xw0_ref[...], preferred_element_type=jnp.float32)
    d = jax.lax.dot_general(b, n.astype(jnp.bfloat16), (((0,), (0,)), ((), ())),
                            preferred_element_type=jnp.float32)

    @pl.when(i == 0)
    def _():
        out_ref[...] = d

    @pl.when(i != 0)
    def _():
        out_ref[...] += d

    @pl.when(i == n_steps - 1)
    def _():
        out_ref[...] = jnp.maximum(out_ref[...] + d2_ref[...] + xw1_ref[...], 0.0)


def kernel(x, B1, B2, W0, W1, W2):
    f32 = jnp.float32
    bf16 = jnp.bfloat16

    xw0, xw1, xw2 = pl.pallas_call(
        _xw_kernel,
        grid=(_N_EDGES // _XB,),
        in_specs=[
            pl.BlockSpec((_XB, _F), lambda i: (i, 0)),
            pl.BlockSpec((_F, _F), lambda i: (0, 0)),
            pl.BlockSpec((_F, _F), lambda i: (0, 0)),
            pl.BlockSpec((_F, _F), lambda i: (0, 0)),
        ],
        out_specs=[
            pl.BlockSpec((_XB, _F), lambda i: (i, 0)),
            pl.BlockSpec((_XB, _F), lambda i: (i, 0)),
            pl.BlockSpec((_XB, _F), lambda i: (i, 0)),
        ],
        out_shape=[
            jax.ShapeDtypeStruct((_N_EDGES, _F), bf16),
            jax.ShapeDtypeStruct((_N_EDGES, _F), f32),
            jax.ShapeDtypeStruct((_N_EDGES, _F), bf16),
        ],
        compiler_params=pltpu.CompilerParams(
            dimension_semantics=("arbitrary",)),
    )(x, W0, W1, W2)

    d2 = pl.pallas_call(
        _d2_kernel,
        grid=(_N_TRI // _JB,),
        in_specs=[
            pl.BlockSpec((_N_EDGES, _JB), lambda j: (0, j)),
            pl.BlockSpec((_N_EDGES, _F), lambda j: (0, 0)),
        ],
        out_specs=pl.BlockSpec((_N_EDGES, _F), lambda j: (0, 0)),
        out_shape=jax.ShapeDtypeStruct((_N_EDGES, _F), f32),
        compiler_params=pltpu.CompilerParams(
            dimension_semantics=("arbitrary",)),
    )(B2, xw2)

    out = pl.pallas_call(
        _d0_kernel,
        grid=(_N_NODES // _IB,),
        in_specs=[
            pl.BlockSpec((_IB, _N_EDGES), lambda i: (i, 0)),
            pl.BlockSpec((_N_EDGES, _F), lambda i: (0, 0)),
            pl.BlockSpec((_N_EDGES, _F), lambda i: (0, 0)),
            pl.BlockSpec((_N_EDGES, _F), lambda i: (0, 0)),
        ],
        out_specs=pl.BlockSpec((_N_EDGES, _F), lambda i: (0, 0)),
        out_shape=jax.ShapeDtypeStruct((_N_EDGES, _F), f32),
        compiler_params=pltpu.CompilerParams(
            dimension_semantics=("arbitrary",)),
    )(B1, xw0, d2, xw1)

    return out


# speedup vs baseline: 1.5793x; 1.1492x over previous
import jax
import jax.numpy as jnp
from jax.experimental import pallas as pl
from jax.experimental.pallas import tpu as pltpu

_N_EDGES = 8192
_N_NODES = 2048
_N_TRI = 4096
_F = 128

_STEPS = 16
_JB = _N_TRI // _STEPS
_IB = _N_NODES // _STEPS
_XB = 1024


def _xw_kernel(x_ref, w0_ref, w1_ref, w2_ref, xw0_ref, xw1_ref, xw2_ref):
    x = x_ref[...]
    xw0_ref[...] = jnp.dot(x, w0_ref[...],
                           preferred_element_type=jnp.float32).astype(jnp.bfloat16)
    xw1_ref[...] = jnp.dot(x, w1_ref[...], preferred_element_type=jnp.float32)
    xw2_ref[...] = jnp.dot(x, w2_ref[...],
                           preferred_element_type=jnp.float32).astype(jnp.bfloat16)


def _chains_kernel(b2_ref, b1_ref, xw0_ref, xw1_ref, xw2_ref, out_ref):
    s = pl.program_id(0)
    n_steps = pl.num_programs(0)

    b2 = b2_ref[...].astype(jnp.bfloat16)
    t = jax.lax.dot_general(b2, xw2_ref[...], (((0,), (0,)), ((), ())),
                            preferred_element_type=jnp.float32)
    d = jnp.dot(b2, t.astype(jnp.bfloat16), preferred_element_type=jnp.float32)

    b1 = b1_ref[...].astype(jnp.bfloat16)
    n = jnp.dot(b1, xw0_ref[...], preferred_element_type=jnp.float32)
    e = jax.lax.dot_general(b1, n.astype(jnp.bfloat16), (((0,), (0,)), ((), ())),
                            preferred_element_type=jnp.float32)

    upd = d + e

    @pl.when(s == 0)
    def _():
        out_ref[...] = upd + xw1_ref[...]

    @pl.when(s != 0)
    def _():
        out_ref[...] += upd

    @pl.when(s == n_steps - 1)
    def _():
        out_ref[...] = jnp.maximum(out_ref[...], 0.0)


def kernel(x, B1, B2, W0, W1, W2):
    f32 = jnp.float32
    bf16 = jnp.bfloat16

    xw0, xw1, xw2 = pl.pallas_call(
        _xw_kernel,
        grid=(_N_EDGES // _XB,),
        in_specs=[
            pl.BlockSpec((_XB, _F), lambda i: (i, 0)),
            pl.BlockSpec((_F, _F), lambda i: (0, 0)),
            pl.BlockSpec((_F, _F), lambda i: (0, 0)),
            pl.BlockSpec((_F, _F), lambda i: (0, 0)),
        ],
        out_specs=[
            pl.BlockSpec((_XB, _F), lambda i: (i, 0)),
            pl.BlockSpec((_XB, _F), lambda i: (i, 0)),
            pl.BlockSpec((_XB, _F), lambda i: (i, 0)),
        ],
        out_shape=[
            jax.ShapeDtypeStruct((_N_EDGES, _F), bf16),
            jax.ShapeDtypeStruct((_N_EDGES, _F), f32),
            jax.ShapeDtypeStruct((_N_EDGES, _F), bf16),
        ],
        compiler_params=pltpu.CompilerParams(
            dimension_semantics=("arbitrary",)),
    )(x, W0, W1, W2)

    out = pl.pallas_call(
        _chains_kernel,
        grid=(_STEPS,),
        in_specs=[
            pl.BlockSpec((_N_EDGES, _JB), lambda s: (0, s)),
            pl.BlockSpec((_IB, _N_EDGES), lambda s: (s, 0)),
            pl.BlockSpec((_N_EDGES, _F), lambda s: (0, 0)),
            pl.BlockSpec((_N_EDGES, _F), lambda s: (0, 0)),
            pl.BlockSpec((_N_EDGES, _F), lambda s: (0, 0)),
        ],
        out_specs=pl.BlockSpec((_N_EDGES, _F), lambda s: (0, 0)),
        out_shape=jax.ShapeDtypeStruct((_N_EDGES, _F), f32),
        compiler_params=pltpu.CompilerParams(
            dimension_semantics=("arbitrary",)),
    )(B2, B1, xw0, xw1, xw2)

    return out


# speedup vs baseline: 1.7425x; 1.1033x over previous
import jax
import jax.numpy as jnp
from jax.experimental import pallas as pl
from jax.experimental.pallas import tpu as pltpu

_N_EDGES = 8192
_N_NODES = 2048
_N_TRI = 4096
_F = 128

_STEPS = 16
_JB = _N_TRI // _STEPS
_IB = _N_NODES // _STEPS


def _scone_kernel(x_ref, w0_ref, w1_ref, w2_ref, b2_ref, b1_ref, out_ref,
                  xw0_s, xw2_s):
    s = pl.program_id(0)
    n_steps = pl.num_programs(0)

    @pl.when(s == 0)
    def _():
        xb = x_ref[...].astype(jnp.bfloat16)
        xw0_s[...] = jnp.dot(xb, w0_ref[...].astype(jnp.bfloat16),
                             preferred_element_type=jnp.float32).astype(jnp.bfloat16)
        xw2_s[...] = jnp.dot(xb, w2_ref[...].astype(jnp.bfloat16),
                             preferred_element_type=jnp.float32).astype(jnp.bfloat16)
        out_ref[...] = jnp.dot(xb, w1_ref[...].astype(jnp.bfloat16),
                               preferred_element_type=jnp.float32)

    b2 = b2_ref[...].astype(jnp.bfloat16)
    t = jax.lax.dot_general(b2, xw2_s[...], (((0,), (0,)), ((), ())),
                            preferred_element_type=jnp.float32)
    d = jnp.dot(b2, t.astype(jnp.bfloat16), preferred_element_type=jnp.float32)

    b1 = b1_ref[...].astype(jnp.bfloat16)
    n = jnp.dot(b1, xw0_s[...], preferred_element_type=jnp.float32)
    e = jax.lax.dot_general(b1, n.astype(jnp.bfloat16), (((0,), (0,)), ((), ())),
                            preferred_element_type=jnp.float32)

    out_ref[...] += d + e

    @pl.when(s == n_steps - 1)
    def _():
        out_ref[...] = jnp.maximum(out_ref[...], 0.0)


def kernel(x, B1, B2, W0, W1, W2):
    return pl.pallas_call(
        _scone_kernel,
        grid=(_STEPS,),
        in_specs=[
            pl.BlockSpec((_N_EDGES, _F), lambda s: (0, 0)),
            pl.BlockSpec((_F, _F), lambda s: (0, 0)),
            pl.BlockSpec((_F, _F), lambda s: (0, 0)),
            pl.BlockSpec((_F, _F), lambda s: (0, 0)),
            pl.BlockSpec((_N_EDGES, _JB), lambda s: (0, s)),
            pl.BlockSpec((_IB, _N_EDGES), lambda s: (s, 0)),
        ],
        out_specs=pl.BlockSpec((_N_EDGES, _F), lambda s: (0, 0)),
        out_shape=jax.ShapeDtypeStruct((_N_EDGES, _F), jnp.float32),
        scratch_shapes=[
            pltpu.VMEM((_N_EDGES, _F), jnp.bfloat16),
            pltpu.VMEM((_N_EDGES, _F), jnp.bfloat16),
        ],
        compiler_params=pltpu.CompilerParams(
            dimension_semantics=("arbitrary",)),
    )(x, W0, W1, W2, B2, B1)
